# Initial kernel scaffold; baseline (speedup 1.0000x reference)
import jax
import jax.numpy as jnp
from jax.experimental import pallas as pl


def kernel(x, beta, isTrack, idx, segment_ids):
    raise NotImplementedError("write your pallas kernel here")


# speedup vs baseline: 4.4099x; 4.4099x over previous
import functools

import jax
import jax.numpy as jnp
from jax.experimental import pallas as pl

_N_ITERS = 8
_T_B = 0.1
_T_D = 0.5
_BLK = 2048
_SEG = 128


def _phase1(beta_ref, trk_ref, asg_ref, sid_ref, segmax_ref, winner_ref):
    b = pl.program_id(0)
    beta = beta_ref[...]
    trk = trk_ref[...]
    asg = asg_ref[...]
    sid = sid_ref[...]
    rem = 1.0 - asg
    mb = beta * rem
    mt = trk * rem
    score = mb * (1.0 + 999.0 * mt)
    seg_iota = jax.lax.broadcasted_iota(jnp.int32, (_BLK, _SEG), 1).astype(jnp.float32)
    mask = sid == seg_iota
    neg = jnp.full((), -jnp.inf, jnp.float32)
    masked = jnp.where(mask, score, neg)
    m = jnp.max(masked, axis=0, keepdims=True)
    node_id = jax.lax.broadcasted_iota(jnp.int32, (_BLK, _SEG), 0).astype(jnp.float32)
    node_id = node_id + (b * _BLK).astype(jnp.float32)
    at_max = jnp.where(mask & (masked >= m), node_id, -1.0)
    w = jnp.max(at_max, axis=0, keepdims=True)

    @pl.when(b == 0)
    def _():
        segmax_ref[...] = m
        winner_ref[...] = w

    @pl.when(b > 0)
    def _():
        cur = segmax_ref[...]
        curw = winner_ref[...]
        better = m > cur
        eq = m == cur
        winner_ref[...] = jnp.where(
            better, w, jnp.where(eq, jnp.maximum(curw, w), curw))
        segmax_ref[...] = jnp.where(better, m, cur)


def _phase2(it, x_ref, sid_ref, beta_ref, trk_ref, idx_ref, asg_ref, clu_ref,
            icp_ref, wx_ref, wb_ref, wi_ref,
            icp_out, clu_out, asg_out, dist_out):
    x = x_ref[...]
    sid = sid_ref[...]
    beta = beta_ref[...]
    trk = trk_ref[...]
    idxf = idx_ref[...]
    asg = asg_ref[...]
    clu = clu_ref[...]
    icp = icp_ref[...]
    wx = wx_ref[...]
    wb = wb_ref[...]
    wi = wi_ref[...]
    seg_iota = jax.lax.broadcasted_iota(jnp.int32, (_BLK, _SEG), 1).astype(jnp.float32)
    onehot = (sid == seg_iota).astype(jnp.float32)
    max_x = jnp.dot(onehot, wx, preferred_element_type=jnp.float32)
    max_beta = jnp.sum(onehot * wb, axis=1, keepdims=True)
    max_idx = jnp.sum(onehot * wi, axis=1, keepdims=True)
    is_w = max_idx == idxf
    cond = is_w & ((max_beta >= _T_B) | (trk == 1.0))
    new_track = is_w & (trk == 1.0)
    icp_out[...] = jnp.where(cond, 1.0, icp)
    diff = max_x - x
    dist = jnp.sqrt(jnp.sum(diff * diff, axis=1, keepdims=True) + 1e-12)
    dist_out[...] = dist
    news = (asg == 0.0) & (dist <= _T_D) & (beta >= _T_B) & (trk == 0.0)
    news = news | new_track
    clu_out[...] = jnp.where(news, float(it), clu)
    asg_out[...] = jnp.where(news, 1.0, asg)


def kernel(x, beta, isTrack, idx, segment_ids):
    n = x.shape[0]
    nb = (n + _BLK - 1) // _BLK
    np_ = nb * _BLK
    pad = np_ - n

    x_p = jnp.pad(x.astype(jnp.float32), ((0, pad), (0, 0)))
    beta_p = jnp.pad(beta.astype(jnp.float32), (0, pad)).reshape(np_, 1)
    trk_p = jnp.pad(isTrack.astype(jnp.float32), (0, pad)).reshape(np_, 1)
    idx_p = jnp.pad(idx.astype(jnp.float32), (0, pad)).reshape(np_, 1)
    sid_p = jnp.pad(segment_ids.astype(jnp.float32), (0, pad),
                    constant_values=float(_SEG - 1)).reshape(np_, 1)

    col = pl.BlockSpec((_BLK, 1), lambda b: (b, 0))
    seg_row = pl.BlockSpec((1, _SEG), lambda b: (0, 0))

    phase1 = pl.pallas_call(
        _phase1,
        grid=(nb,),
        in_specs=[col, col, col, col],
        out_specs=(seg_row, seg_row),
        out_shape=(jax.ShapeDtypeStruct((1, _SEG), jnp.float32),
                   jax.ShapeDtypeStruct((1, _SEG), jnp.float32)),
    )

    col_f32 = jax.ShapeDtypeStruct((np_, 1), jnp.float32)
    phase2_calls = [
        pl.pallas_call(
            functools.partial(_phase2, it),
            grid=(nb,),
            in_specs=[pl.BlockSpec((_BLK, 128), lambda b: (b, 0)),
                      col, col, col, col, col, col, col,
                      pl.BlockSpec((_SEG, 128), lambda b: (0, 0)),
                      seg_row, seg_row],
            out_specs=(col, col, col, col),
            out_shape=(col_f32, col_f32, col_f32, col_f32),
        )
        for it in range(_N_ITERS)
    ]

    assigned = jnp.zeros((np_, 1), jnp.float32)
    clu = jnp.full((np_, 1), -1.0, jnp.float32)
    icp = jnp.zeros((np_, 1), jnp.float32)
    dist = jnp.zeros((np_, 1), jnp.float32)

    for it in range(_N_ITERS):
        _, winner = phase1(beta_p, trk_p, assigned, sid_p)
        wi_idx = jnp.maximum(winner.reshape(_SEG).astype(jnp.int32), 0)
        w_x = x_p[wi_idx]
        msg_beta = (beta_p * (1.0 - assigned)).reshape(np_)
        w_beta = msg_beta[wi_idx].reshape(1, _SEG)
        w_idx = idx_p.reshape(np_)[wi_idx].reshape(1, _SEG)
        icp, clu, assigned, dist = phase2_calls[it](
            x_p, sid_p, beta_p, trk_p, idx_p, assigned, clu, icp,
            w_x, w_beta, w_idx)

    return (icp[:n, 0], clu[:n, 0], assigned[:n, 0], dist[:n, 0])
